# Initial kernel scaffold; baseline (speedup 1.0000x reference)
import jax
import jax.numpy as jnp
from jax.experimental import pallas as pl


def kernel(input_embeddings, edge_index, W_node, b_node, W_edge, b_edge, W_upd, b_upd):
    raise NotImplementedError("write your pallas kernel here")


# speedup vs baseline: 6.1420x; 6.1420x over previous
import functools

import jax
import jax.numpy as jnp
from jax import lax
from jax.experimental import pallas as pl
from jax.experimental.pallas import tpu as pltpu
from jax.experimental.pallas import tpu_sc as plsc

D = 128
N = 10000
NTILES = 16
L = 16
RPT = 640
N_PAD = NTILES * RPT
CHUNK = 128
G = 8
R_BLK = 1000


def _tc1_body(x_ref, wn_ref, bn_ref, we_ref, be_ref, t_ref, a2_ref):
    node = jnp.dot(x_ref[...], wn_ref[...], preferred_element_type=jnp.float32)
    node = node + bn_ref[...]
    a2 = jnp.dot(node, we_ref[:D], preferred_element_type=jnp.float32) + be_ref[...]
    bt = jnp.dot(node, we_ref[D:], preferred_element_type=jnp.float32)
    t_ref[0] = node
    t_ref[1] = bt
    a2_ref[...] = a2


def _tc2_body(t_ref, s_ref, deg_ref, a2_ref, wu_ref, bu_ref, o_ref):
    node = t_ref[0]
    s_n = s_ref[0]
    s_b = s_ref[1]
    agg_e = deg_ref[...] * a2_ref[...] + s_b

    def g(v):
        return 0.5 * v * (1.0 + lax.erf(v * (2.0 ** -0.5)))

    out = jnp.dot(g(node), wu_ref[:D], preferred_element_type=jnp.float32)
    out = out + jnp.dot(g(s_n), wu_ref[D:2 * D], preferred_element_type=jnp.float32)
    out = out + jnp.dot(g(agg_e), wu_ref[2 * D:], preferred_element_type=jnp.float32)
    o_ref[...] = out + bu_ref[...]


@functools.lru_cache(maxsize=None)
def _make_sc_scatter(n_groups: int):
    mesh = plsc.VectorSubcoreMesh(core_axis_name="c", subcore_axis_name="s")

    def body(t_hbm, src_hbm, dst_hbm, s_hbm, deg_hbm,
             idx_src, idx_dst, rows, ones_b, zb1, acc, deg_sh,
             gsem, ssem, dsem):
        cid = lax.axis_index("c")
        tid = lax.axis_index("s")
        base = tid * RPT
        zeros16 = jnp.zeros((L,), jnp.float32)
        ones16 = jnp.ones((L,), jnp.float32)

        def init_ones(i, carry):
            ones_b[pl.ds(i * L, L)] = ones16
            return carry
        lax.fori_loop(0, CHUNK // L, init_ones, 0)

        def zero_zb1(i, carry):
            zb1[pl.ds(i * L, L)] = zeros16
            return carry
        lax.fori_loop(0, RPT // L, zero_zb1, 0)

        def zero_rows(i, carry):
            for k in range(D // L):
                rows[0, i, pl.ds(k * L, L)] = zeros16
            return carry
        lax.fori_loop(0, CHUNK, zero_rows, 0)

        for q in range(RPT // CHUNK):
            pltpu.sync_copy(rows.at[0],
                            acc.at[pl.ds(base + q * CHUNK, CHUNK)])
        pltpu.sync_copy(zb1, deg_sh.at[pl.ds(base, RPT)])

        pltpu.sync_copy(src_hbm.at[tid, 0], idx_src.at[0])
        pltpu.sync_copy(dst_hbm.at[cid, tid, 0], idx_dst.at[0])

        plsc.subcore_barrier()

        pltpu.async_copy(t_hbm.at[idx_dst.at[0].at[0]], rows.at[0], gsem)

        def group(g, carry):
            b = lax.bitwise_and(g, 1)
            nb = lax.bitwise_xor(b, 1)

            @pl.when(g + 1 < n_groups)
            def _():
                pltpu.async_copy(src_hbm.at[tid, g + 1], idx_src.at[nb], ssem)
                pltpu.async_copy(dst_hbm.at[cid, tid, g + 1],
                                 idx_dst.at[nb], dsem)

            for j in range(G):
                if j + 1 < G:
                    pltpu.async_copy(t_hbm.at[idx_dst.at[b].at[j + 1]],
                                     rows.at[(j + 1) % 2], gsem)
                else:
                    @pl.when(g + 1 < n_groups)
                    def _():
                        pltpu.make_async_copy(src_hbm.at[tid, g + 1],
                                              idx_src.at[nb], ssem).wait()
                        pltpu.make_async_copy(dst_hbm.at[cid, tid, g + 1],
                                              idx_dst.at[nb], dsem).wait()
                        pltpu.async_copy(t_hbm.at[idx_dst.at[nb].at[0]],
                                         rows.at[0], gsem)
                pltpu.make_async_copy(t_hbm.at[idx_dst.at[b].at[j]],
                                      rows.at[j % 2], gsem).wait()
                pltpu.sync_copy(rows.at[j % 2],
                                acc.at[idx_src.at[b].at[j]], add=True)

                @pl.when(cid == 0)
                def _():
                    pltpu.sync_copy(ones_b, deg_sh.at[idx_src.at[b].at[j]],
                                    add=True)
            return carry
        lax.fori_loop(0, n_groups, group, 0)

        plsc.subcore_barrier()

        pltpu.sync_copy(acc.at[pl.ds(base, RPT)],
                        s_hbm.at[cid, pl.ds(base, RPT)])

        @pl.when(cid == 0)
        def _():
            pltpu.sync_copy(deg_sh.at[pl.ds(base, RPT)],
                            deg_hbm.at[pl.ds(base, RPT)])

    return pl.kernel(
        body,
        out_type=(
            jax.ShapeDtypeStruct((2, N_PAD, D), jnp.float32),
            jax.ShapeDtypeStruct((N_PAD,), jnp.float32),
        ),
        mesh=mesh,
        compiler_params=pltpu.CompilerParams(needs_layout_passes=False),
        scratch_types=[
            pltpu.VMEM((2, G, CHUNK), jnp.int32),
            pltpu.VMEM((2, G, CHUNK), jnp.int32),
            pltpu.VMEM((2, CHUNK, D), jnp.float32),
            pltpu.VMEM((CHUNK,), jnp.float32),
            pltpu.VMEM((RPT,), jnp.float32),
            pltpu.VMEM_SHARED((N_PAD, D), jnp.float32),
            pltpu.VMEM_SHARED((N_PAD,), jnp.float32),
            pltpu.SemaphoreType.DMA,
            pltpu.SemaphoreType.DMA,
            pltpu.SemaphoreType.DMA,
        ],
    )


def kernel(input_embeddings, edge_index, W_node, b_node, W_edge, b_edge,
           W_upd, b_upd):
    x = input_embeddings
    src = edge_index[0].astype(jnp.int32)
    dst = edge_index[1].astype(jnp.int32)
    e = src.shape[0]
    epg = NTILES * G * CHUNK
    n_groups = -(-e // epg)
    pad = n_groups * epg - e
    if pad:
        src = jnp.concatenate([src, jnp.full((pad,), N, jnp.int32)])
        dst = jnp.concatenate([dst, jnp.zeros((pad,), jnp.int32)])
    src4 = src.reshape(NTILES, n_groups, G, CHUNK)
    dst5 = jnp.stack([dst, dst + N]).reshape(2, NTILES, n_groups, G, CHUNK)

    bn = b_node.reshape(1, D)
    be = b_edge.reshape(1, D)
    bu = b_upd.reshape(1, D)

    grid = N // R_BLK
    t3, a2 = pl.pallas_call(
        _tc1_body,
        grid=(grid,),
        in_specs=[
            pl.BlockSpec((R_BLK, D), lambda i: (i, 0)),
            pl.BlockSpec((D, D), lambda i: (0, 0)),
            pl.BlockSpec((1, D), lambda i: (0, 0)),
            pl.BlockSpec((2 * D, D), lambda i: (0, 0)),
            pl.BlockSpec((1, D), lambda i: (0, 0)),
        ],
        out_specs=[
            pl.BlockSpec((2, R_BLK, D), lambda i: (0, i, 0)),
            pl.BlockSpec((R_BLK, D), lambda i: (i, 0)),
        ],
        out_shape=[
            jax.ShapeDtypeStruct((2, N, D), jnp.float32),
            jax.ShapeDtypeStruct((N, D), jnp.float32),
        ],
    )(x, W_node, bn, W_edge, be)

    table = t3.reshape(2 * N, D)
    s, deg = _make_sc_scatter(n_groups)(table, src4, dst5)
    deg2 = deg.reshape(N_PAD, 1)

    out = pl.pallas_call(
        _tc2_body,
        grid=(grid,),
        in_specs=[
            pl.BlockSpec((1, R_BLK, D), lambda i: (0, i, 0)),
            pl.BlockSpec((2, R_BLK, D), lambda i: (0, i, 0)),
            pl.BlockSpec((R_BLK, 1), lambda i: (i, 0)),
            pl.BlockSpec((R_BLK, D), lambda i: (i, 0)),
            pl.BlockSpec((3 * D, D), lambda i: (0, 0)),
            pl.BlockSpec((1, D), lambda i: (0, 0)),
        ],
        out_specs=pl.BlockSpec((R_BLK, D), lambda i: (i, 0)),
        out_shape=jax.ShapeDtypeStruct((N, D), jnp.float32),
    )(t3, s, deg2, a2, W_upd, bu)
    return out


# speedup vs baseline: 6.2442x; 1.0166x over previous
import functools

import jax
import jax.numpy as jnp
from jax import lax
from jax.experimental import pallas as pl
from jax.experimental.pallas import tpu as pltpu
from jax.experimental.pallas import tpu_sc as plsc

D = 128
N = 10000
NTILES = 16
L = 16
RPT = 640
N_PAD = NTILES * RPT
CHUNK = 128
G = 8
R_BLK = 1000


def _tc1_body(x_ref, wn_ref, bn_ref, we_ref, be_ref, t_ref, a2_ref):
    node = jnp.dot(x_ref[...], wn_ref[...], preferred_element_type=jnp.float32)
    node = node + bn_ref[...]
    a2 = jnp.dot(node, we_ref[:D], preferred_element_type=jnp.float32) + be_ref[...]
    bt = jnp.dot(node, we_ref[D:], preferred_element_type=jnp.float32)
    t_ref[0] = node
    t_ref[1] = bt
    a2_ref[...] = a2


def _tc2_body(t_ref, s_ref, deg_ref, a2_ref, wu_ref, bu_ref, o_ref):
    node = t_ref[0]
    s_n = s_ref[0]
    s_b = s_ref[1]
    agg_e = deg_ref[...] * a2_ref[...] + s_b

    def g(v):
        return 0.5 * v * (1.0 + lax.erf(v * (2.0 ** -0.5)))

    out = jnp.dot(g(node), wu_ref[:D], preferred_element_type=jnp.float32)
    out = out + jnp.dot(g(s_n), wu_ref[D:2 * D], preferred_element_type=jnp.float32)
    out = out + jnp.dot(g(agg_e), wu_ref[2 * D:], preferred_element_type=jnp.float32)
    o_ref[...] = out + bu_ref[...]


@functools.lru_cache(maxsize=None)
def _make_sc_scatter(n_groups: int):
    mesh = plsc.VectorSubcoreMesh(core_axis_name="c", subcore_axis_name="s")

    def body(t_hbm, src_hbm, dst_hbm, s_hbm, deg_hbm,
             idx_src, idx_dst, rows, ones_b, zb1, acc, deg_sh,
             gsem, ssem, dsem, csem):
        cid = lax.axis_index("c")
        tid = lax.axis_index("s")
        base = tid * RPT
        zeros16 = jnp.zeros((L,), jnp.float32)
        ones16 = jnp.ones((L,), jnp.float32)

        def init_ones(i, carry):
            ones_b[pl.ds(i * L, L)] = ones16
            return carry
        lax.fori_loop(0, CHUNK // L, init_ones, 0)

        def zero_zb1(i, carry):
            zb1[pl.ds(i * L, L)] = zeros16
            return carry
        lax.fori_loop(0, RPT // L, zero_zb1, 0)

        def zero_rows(i, carry):
            for k in range(D // L):
                rows[0, i, pl.ds(k * L, L)] = zeros16
            return carry
        lax.fori_loop(0, CHUNK, zero_rows, 0)

        for q in range(RPT // CHUNK):
            pltpu.sync_copy(rows.at[0],
                            acc.at[pl.ds(base + q * CHUNK, CHUNK)])
        pltpu.sync_copy(zb1, deg_sh.at[pl.ds(base, RPT)])

        pltpu.sync_copy(src_hbm.at[tid, 0], idx_src.at[0])
        pltpu.sync_copy(dst_hbm.at[cid, tid, 0], idx_dst.at[0])

        plsc.subcore_barrier()

        pltpu.async_copy(t_hbm.at[idx_dst.at[0].at[0]], rows.at[0], gsem)

        def group(g, carry):
            b = lax.bitwise_and(g, 1)
            nb = lax.bitwise_xor(b, 1)

            for j in range(G):
                if j == 2:
                    @pl.when(g + 1 < n_groups)
                    def _():
                        pltpu.async_copy(src_hbm.at[tid, g + 1],
                                         idx_src.at[nb], ssem)
                        pltpu.async_copy(dst_hbm.at[cid, tid, g + 1],
                                         idx_dst.at[nb], dsem)
                if j >= 2:
                    pltpu.make_async_copy(rows.at[(j - 1) % 2],
                                          acc.at[idx_src.at[b].at[j - 1]],
                                          csem).wait()
                elif j == 1:
                    @pl.when(g > 0)
                    def _():
                        pltpu.make_async_copy(rows.at[0],
                                              acc.at[idx_src.at[b].at[0]],
                                              csem).wait()
                else:
                    @pl.when(g > 0)
                    def _():
                        pltpu.make_async_copy(rows.at[1],
                                              acc.at[idx_src.at[nb].at[G - 1]],
                                              csem).wait()
                if j + 1 < G:
                    pltpu.async_copy(t_hbm.at[idx_dst.at[b].at[j + 1]],
                                     rows.at[(j + 1) % 2], gsem)
                else:
                    @pl.when(g + 1 < n_groups)
                    def _():
                        pltpu.make_async_copy(src_hbm.at[tid, g + 1],
                                              idx_src.at[nb], ssem).wait()
                        pltpu.make_async_copy(dst_hbm.at[cid, tid, g + 1],
                                              idx_dst.at[nb], dsem).wait()
                        pltpu.async_copy(t_hbm.at[idx_dst.at[nb].at[0]],
                                         rows.at[0], gsem)
                pltpu.make_async_copy(t_hbm.at[idx_dst.at[b].at[j]],
                                      rows.at[j % 2], gsem).wait()
                pltpu.async_copy(rows.at[j % 2],
                                 acc.at[idx_src.at[b].at[j]], csem, add=True)

                @pl.when(cid == 0)
                def _():
                    pltpu.sync_copy(ones_b, deg_sh.at[idx_src.at[b].at[j]],
                                    add=True)
            return carry
        lax.fori_loop(0, n_groups, group, 0)

        lb = lax.bitwise_and(n_groups - 1, 1)
        pltpu.make_async_copy(rows.at[(G - 1) % 2],
                              acc.at[idx_src.at[lb].at[G - 1]], csem).wait()

        plsc.subcore_barrier()

        pltpu.sync_copy(acc.at[pl.ds(base, RPT)],
                        s_hbm.at[cid, pl.ds(base, RPT)])

        @pl.when(cid == 0)
        def _():
            pltpu.sync_copy(deg_sh.at[pl.ds(base, RPT)],
                            deg_hbm.at[pl.ds(base, RPT)])

    return pl.kernel(
        body,
        out_type=(
            jax.ShapeDtypeStruct((2, N_PAD, D), jnp.float32),
            jax.ShapeDtypeStruct((N_PAD,), jnp.float32),
        ),
        mesh=mesh,
        compiler_params=pltpu.CompilerParams(needs_layout_passes=False),
        scratch_types=[
            pltpu.VMEM((2, G, CHUNK), jnp.int32),
            pltpu.VMEM((2, G, CHUNK), jnp.int32),
            pltpu.VMEM((2, CHUNK, D), jnp.float32),
            pltpu.VMEM((CHUNK,), jnp.float32),
            pltpu.VMEM((RPT,), jnp.float32),
            pltpu.VMEM_SHARED((N_PAD, D), jnp.float32),
            pltpu.VMEM_SHARED((N_PAD,), jnp.float32),
            pltpu.SemaphoreType.DMA,
            pltpu.SemaphoreType.DMA,
            pltpu.SemaphoreType.DMA,
            pltpu.SemaphoreType.DMA,
        ],
    )


def kernel(input_embeddings, edge_index, W_node, b_node, W_edge, b_edge,
           W_upd, b_upd):
    x = input_embeddings
    src = edge_index[0].astype(jnp.int32)
    dst = edge_index[1].astype(jnp.int32)
    e = src.shape[0]
    epg = NTILES * G * CHUNK
    n_groups = -(-e // epg)
    pad = n_groups * epg - e
    if pad:
        src = jnp.concatenate([src, jnp.full((pad,), N, jnp.int32)])
        dst = jnp.concatenate([dst, jnp.zeros((pad,), jnp.int32)])
    src4 = src.reshape(NTILES, n_groups, G, CHUNK)
    dst5 = jnp.stack([dst, dst + N]).reshape(2, NTILES, n_groups, G, CHUNK)

    bn = b_node.reshape(1, D)
    be = b_edge.reshape(1, D)
    bu = b_upd.reshape(1, D)

    grid = N // R_BLK
    t3, a2 = pl.pallas_call(
        _tc1_body,
        grid=(grid,),
        in_specs=[
            pl.BlockSpec((R_BLK, D), lambda i: (i, 0)),
            pl.BlockSpec((D, D), lambda i: (0, 0)),
            pl.BlockSpec((1, D), lambda i: (0, 0)),
            pl.BlockSpec((2 * D, D), lambda i: (0, 0)),
            pl.BlockSpec((1, D), lambda i: (0, 0)),
        ],
        out_specs=[
            pl.BlockSpec((2, R_BLK, D), lambda i: (0, i, 0)),
            pl.BlockSpec((R_BLK, D), lambda i: (i, 0)),
        ],
        out_shape=[
            jax.ShapeDtypeStruct((2, N, D), jnp.float32),
            jax.ShapeDtypeStruct((N, D), jnp.float32),
        ],
    )(x, W_node, bn, W_edge, be)

    table = t3.reshape(2 * N, D)
    s, deg = _make_sc_scatter(n_groups)(table, src4, dst5)
    deg2 = deg.reshape(N_PAD, 1)

    out = pl.pallas_call(
        _tc2_body,
        grid=(grid,),
        in_specs=[
            pl.BlockSpec((1, R_BLK, D), lambda i: (0, i, 0)),
            pl.BlockSpec((2, R_BLK, D), lambda i: (0, i, 0)),
            pl.BlockSpec((R_BLK, 1), lambda i: (i, 0)),
            pl.BlockSpec((R_BLK, D), lambda i: (i, 0)),
            pl.BlockSpec((3 * D, D), lambda i: (0, 0)),
            pl.BlockSpec((1, D), lambda i: (0, 0)),
        ],
        out_specs=pl.BlockSpec((R_BLK, D), lambda i: (i, 0)),
        out_shape=jax.ShapeDtypeStruct((N, D), jnp.float32),
    )(t3, s, deg2, a2, W_upd, bu)
    return out


# speedup vs baseline: 6.2460x; 1.0003x over previous
import functools

import jax
import jax.numpy as jnp
from jax import lax
from jax.experimental import pallas as pl
from jax.experimental.pallas import tpu as pltpu
from jax.experimental.pallas import tpu_sc as plsc

D = 128
N = 10000
NTILES = 16
L = 16
RPT = 640
N_PAD = NTILES * RPT
CHUNK = 128
G = 8
R_BLK = 1000


def _tc1_body(x_ref, wn_ref, bn_ref, we_ref, be_ref, t_ref, a2_ref):
    node = jnp.dot(x_ref[...], wn_ref[...], preferred_element_type=jnp.float32)
    node = node + bn_ref[...]
    a2 = jnp.dot(node, we_ref[:D], preferred_element_type=jnp.float32) + be_ref[...]
    bt = jnp.dot(node, we_ref[D:], preferred_element_type=jnp.float32)
    t_ref[0] = node
    t_ref[1] = bt
    a2_ref[...] = a2


def _tc2_body(t_ref, s_ref, deg_ref, a2_ref, wu_ref, bu_ref, o_ref):
    node = t_ref[0]
    s_n = s_ref[0]
    s_b = s_ref[1]
    agg_e = deg_ref[...] * a2_ref[...] + s_b

    def g(v):
        return 0.5 * v * (1.0 + lax.erf(v * (2.0 ** -0.5)))

    out = jnp.dot(g(node), wu_ref[:D], preferred_element_type=jnp.float32)
    out = out + jnp.dot(g(s_n), wu_ref[D:2 * D], preferred_element_type=jnp.float32)
    out = out + jnp.dot(g(agg_e), wu_ref[2 * D:], preferred_element_type=jnp.float32)
    o_ref[...] = out + bu_ref[...]


@functools.lru_cache(maxsize=None)
def _make_sc_scatter(n_groups: int):
    mesh = plsc.VectorSubcoreMesh(core_axis_name="c", subcore_axis_name="s")

    def body(t_hbm, src_hbm, dst_hbm, s_hbm, deg_hbm,
             idx_src, idx_dst, rows, ones_b, zb1, acc, deg_sh,
             gsem, ssem, dsem, csem):
        cid = lax.axis_index("c")
        tid = lax.axis_index("s")
        base = tid * RPT
        zeros16 = jnp.zeros((L,), jnp.float32)
        ones16 = jnp.ones((L,), jnp.float32)

        def init_ones(i, carry):
            ones_b[pl.ds(i * L, L)] = ones16
            return carry
        lax.fori_loop(0, CHUNK // L, init_ones, 0)

        def zero_zb1(i, carry):
            zb1[pl.ds(i * L, L)] = zeros16
            return carry
        lax.fori_loop(0, RPT // L, zero_zb1, 0)

        def zero_rows(i, carry):
            for k in range(D // L):
                rows[0, i, pl.ds(k * L, L)] = zeros16
            return carry
        lax.fori_loop(0, CHUNK, zero_rows, 0)

        for q in range(RPT // CHUNK):
            pltpu.sync_copy(rows.at[0],
                            acc.at[pl.ds(base + q * CHUNK, CHUNK)])
        pltpu.sync_copy(zb1, deg_sh.at[pl.ds(base, RPT)])

        pltpu.sync_copy(src_hbm.at[tid, 0], idx_src.at[0])
        pltpu.sync_copy(dst_hbm.at[cid, tid, 0], idx_dst.at[0])

        plsc.subcore_barrier()

        pltpu.async_copy(t_hbm.at[idx_dst.at[0].at[0].at[pl.ds(0, CHUNK // 2)]],
                         rows.at[0].at[pl.ds(0, CHUNK // 2)], gsem)
        pltpu.async_copy(t_hbm.at[idx_dst.at[0].at[0].at[pl.ds(CHUNK // 2, CHUNK // 2)]],
                         rows.at[0].at[pl.ds(CHUNK // 2, CHUNK // 2)], gsem)

        def group(g, carry):
            b = lax.bitwise_and(g, 1)
            nb = lax.bitwise_xor(b, 1)

            for j in range(G):
                if j == 2:
                    @pl.when(g + 1 < n_groups)
                    def _():
                        pltpu.async_copy(src_hbm.at[tid, g + 1],
                                         idx_src.at[nb], ssem)
                        pltpu.async_copy(dst_hbm.at[cid, tid, g + 1],
                                         idx_dst.at[nb], dsem)
                if j >= 2:
                    pltpu.make_async_copy(rows.at[(j - 1) % 2],
                                          acc.at[idx_src.at[b].at[j - 1]],
                                          csem).wait()
                elif j == 1:
                    @pl.when(g > 0)
                    def _():
                        pltpu.make_async_copy(rows.at[0],
                                              acc.at[idx_src.at[b].at[0]],
                                              csem).wait()
                else:
                    @pl.when(g > 0)
                    def _():
                        pltpu.make_async_copy(rows.at[1],
                                              acc.at[idx_src.at[nb].at[G - 1]],
                                              csem).wait()
                if j + 1 < G:
                    for h in range(2):
                        pltpu.async_copy(
                            t_hbm.at[idx_dst.at[b].at[j + 1].at[pl.ds(h * (CHUNK // 2), CHUNK // 2)]],
                            rows.at[(j + 1) % 2].at[pl.ds(h * (CHUNK // 2), CHUNK // 2)],
                            gsem)
                else:
                    @pl.when(g + 1 < n_groups)
                    def _():
                        pltpu.make_async_copy(src_hbm.at[tid, g + 1],
                                              idx_src.at[nb], ssem).wait()
                        pltpu.make_async_copy(dst_hbm.at[cid, tid, g + 1],
                                              idx_dst.at[nb], dsem).wait()
                        for h in range(2):
                            pltpu.async_copy(
                                t_hbm.at[idx_dst.at[nb].at[0].at[pl.ds(h * (CHUNK // 2), CHUNK // 2)]],
                                rows.at[0].at[pl.ds(h * (CHUNK // 2), CHUNK // 2)],
                                gsem)
                for h in range(2):
                    pltpu.make_async_copy(
                        t_hbm.at[idx_dst.at[b].at[j].at[pl.ds(h * (CHUNK // 2), CHUNK // 2)]],
                        rows.at[j % 2].at[pl.ds(h * (CHUNK // 2), CHUNK // 2)],
                        gsem).wait()
                pltpu.async_copy(rows.at[j % 2],
                                 acc.at[idx_src.at[b].at[j]], csem, add=True)

                @pl.when(cid == 0)
                def _():
                    pltpu.sync_copy(ones_b, deg_sh.at[idx_src.at[b].at[j]],
                                    add=True)
            return carry
        lax.fori_loop(0, n_groups, group, 0)

        lb = lax.bitwise_and(n_groups - 1, 1)
        pltpu.make_async_copy(rows.at[(G - 1) % 2],
                              acc.at[idx_src.at[lb].at[G - 1]], csem).wait()

        plsc.subcore_barrier()

        pltpu.sync_copy(acc.at[pl.ds(base, RPT)],
                        s_hbm.at[cid, pl.ds(base, RPT)])

        @pl.when(cid == 0)
        def _():
            pltpu.sync_copy(deg_sh.at[pl.ds(base, RPT)],
                            deg_hbm.at[pl.ds(base, RPT)])

    return pl.kernel(
        body,
        out_type=(
            jax.ShapeDtypeStruct((2, N_PAD, D), jnp.float32),
            jax.ShapeDtypeStruct((N_PAD,), jnp.float32),
        ),
        mesh=mesh,
        compiler_params=pltpu.CompilerParams(needs_layout_passes=False),
        scratch_types=[
            pltpu.VMEM((2, G, CHUNK), jnp.int32),
            pltpu.VMEM((2, G, CHUNK), jnp.int32),
            pltpu.VMEM((2, CHUNK, D), jnp.float32),
            pltpu.VMEM((CHUNK,), jnp.float32),
            pltpu.VMEM((RPT,), jnp.float32),
            pltpu.VMEM_SHARED((N_PAD, D), jnp.float32),
            pltpu.VMEM_SHARED((N_PAD,), jnp.float32),
            pltpu.SemaphoreType.DMA,
            pltpu.SemaphoreType.DMA,
            pltpu.SemaphoreType.DMA,
            pltpu.SemaphoreType.DMA,
        ],
    )


def kernel(input_embeddings, edge_index, W_node, b_node, W_edge, b_edge,
           W_upd, b_upd):
    x = input_embeddings
    src = edge_index[0].astype(jnp.int32)
    dst = edge_index[1].astype(jnp.int32)
    e = src.shape[0]
    epg = NTILES * G * CHUNK
    n_groups = -(-e // epg)
    pad = n_groups * epg - e
    if pad:
        src = jnp.concatenate([src, jnp.full((pad,), N, jnp.int32)])
        dst = jnp.concatenate([dst, jnp.zeros((pad,), jnp.int32)])
    src4 = src.reshape(NTILES, n_groups, G, CHUNK)
    dst5 = jnp.stack([dst, dst + N]).reshape(2, NTILES, n_groups, G, CHUNK)

    bn = b_node.reshape(1, D)
    be = b_edge.reshape(1, D)
    bu = b_upd.reshape(1, D)

    grid = N // R_BLK
    t3, a2 = pl.pallas_call(
        _tc1_body,
        grid=(grid,),
        in_specs=[
            pl.BlockSpec((R_BLK, D), lambda i: (i, 0)),
            pl.BlockSpec((D, D), lambda i: (0, 0)),
            pl.BlockSpec((1, D), lambda i: (0, 0)),
            pl.BlockSpec((2 * D, D), lambda i: (0, 0)),
            pl.BlockSpec((1, D), lambda i: (0, 0)),
        ],
        out_specs=[
            pl.BlockSpec((2, R_BLK, D), lambda i: (0, i, 0)),
            pl.BlockSpec((R_BLK, D), lambda i: (i, 0)),
        ],
        out_shape=[
            jax.ShapeDtypeStruct((2, N, D), jnp.float32),
            jax.ShapeDtypeStruct((N, D), jnp.float32),
        ],
    )(x, W_node, bn, W_edge, be)

    table = t3.reshape(2 * N, D)
    s, deg = _make_sc_scatter(n_groups)(table, src4, dst5)
    deg2 = deg.reshape(N_PAD, 1)

    out = pl.pallas_call(
        _tc2_body,
        grid=(grid,),
        in_specs=[
            pl.BlockSpec((1, R_BLK, D), lambda i: (0, i, 0)),
            pl.BlockSpec((2, R_BLK, D), lambda i: (0, i, 0)),
            pl.BlockSpec((R_BLK, 1), lambda i: (i, 0)),
            pl.BlockSpec((R_BLK, D), lambda i: (i, 0)),
            pl.BlockSpec((3 * D, D), lambda i: (0, 0)),
            pl.BlockSpec((1, D), lambda i: (0, 0)),
        ],
        out_specs=pl.BlockSpec((R_BLK, D), lambda i: (i, 0)),
        out_shape=jax.ShapeDtypeStruct((N, D), jnp.float32),
    )(t3, s, deg2, a2, W_upd, bu)
    return out
